# baseline (speedup 1.0000x reference)
import functools

import jax
import jax.numpy as jnp
from jax import lax
from jax.experimental import pallas as pl
from jax.experimental.pallas import tpu as pltpu
import jax.experimental.pallas.tpu_sc as plsc

N = 10000
E = 320000
IN_DIM = 128
HID_DIM = 64
EMB_DIM = 16

NUM_CORES = 2
NUM_SUBCORES = 16
NW = NUM_CORES * NUM_SUBCORES
CH = 128
NCH = 79
EPW = CH * NCH
E_PAD = NW * EPW
NCH_EP = (79, 79)
NACC = 10240
RPT = NACC // NUM_SUBCORES
PAD_COL = NACC - 1


def _sc_mesh():
    return plsc.VectorSubcoreMesh(
        core_axis_name="c", subcore_axis_name="s",
        num_cores=NUM_CORES, num_subcores=NUM_SUBCORES)


def _sc_degree(col_t, ones_vec):
    @functools.partial(
        pl.kernel,
        out_type=jax.ShapeDtypeStruct((NUM_CORES * NACC,), jnp.float32),
        mesh=_sc_mesh(),
        scratch_types=[
            pltpu.VMEM((NCH, CH), jnp.int32),
            pltpu.VMEM((CH,), jnp.float32),
            pltpu.VMEM((RPT,), jnp.float32),
            pltpu.VMEM_SHARED((NACC,), jnp.float32),
        ],
    )
    def deg_kernel(col_hbm, ones_hbm, out_hbm, idx_v, ones_v, zer_v, hist_sh):
        cid = lax.axis_index("c")
        sid = lax.axis_index("s")
        wid = sid * NUM_CORES + cid
        pltpu.sync_copy(col_hbm.at[wid], idx_v)
        pltpu.sync_copy(ones_hbm, ones_v)

        zero16 = jnp.zeros((16,), jnp.float32)

        def zbody(i, carry):
            zer_v[pl.ds(pl.multiple_of(i * 16, 16), 16)] = zero16
            return carry

        lax.fori_loop(0, RPT // 16, zbody, 0)
        base = pl.multiple_of(sid * RPT, 128)
        pltpu.sync_copy(zer_v, hist_sh.at[pl.ds(base, RPT)])
        plsc.subcore_barrier()

        def body(ci, carry):
            pltpu.sync_copy(ones_v, hist_sh.at[idx_v.at[ci]], add=True)
            return carry

        lax.fori_loop(0, NCH, body, 0)
        plsc.subcore_barrier()
        obase = pl.multiple_of(cid * NACC + sid * RPT, 128)
        pltpu.sync_copy(hist_sh.at[pl.ds(base, RPT)],
                        out_hbm.at[pl.ds(obase, RPT)])

    return deg_kernel(col_t, ones_vec)


def _sc_edge_pass(row_t, col_t, table, d, nchs, stage):
    nch_max = max(nchs)
    if stage:
        scratch = [
            pltpu.VMEM((8, CH), jnp.int32),
            pltpu.VMEM((8, CH), jnp.int32),
            pltpu.VMEM((4, CH, d), jnp.float32),
            pltpu.VMEM_SHARED((NACC, d), jnp.float32),
            pltpu.VMEM_SHARED((NACC, d), jnp.float32),
            pltpu.SemaphoreType.DMA,
            pltpu.SemaphoreType.DMA,
            pltpu.SemaphoreType.DMA,
        ]
    else:
        scratch = [
            pltpu.VMEM((nch_max, CH), jnp.int32),
            pltpu.VMEM((nch_max, CH), jnp.int32),
            pltpu.VMEM((6, CH, d), jnp.float32),
            pltpu.VMEM_SHARED((NACC, d), jnp.float32),
            pltpu.SemaphoreType.DMA,
            pltpu.SemaphoreType.DMA,
        ]

    @functools.partial(
        pl.kernel,
        out_type=jax.ShapeDtypeStruct((NUM_CORES, NACC, d), jnp.float32),
        mesh=_sc_mesh(),
        scratch_types=scratch,
        compiler_params=pltpu.CompilerParams(use_tc_tiling_on_sc=False),
    )
    def edge_kernel(row_hbm, col_hbm, table_hbm, out_hbm, *refs):
        cid = lax.axis_index("c")
        sid = lax.axis_index("s")
        wid = sid * NUM_CORES + cid
        nch = jnp.where(cid == 0, nchs[0], nchs[1])
        base = pl.multiple_of(sid * RPT, 8)
        zero16 = jnp.zeros((16,), jnp.float32)

        if stage:
            idx_r, idx_c, buf, acc_sh, tab_sh, gsem, ssem, isem = refs
            for p in range(7):
                @pl.when(p < nch)
                def _():
                    pltpu.async_copy(row_hbm.at[wid, p], idx_r.at[p], isem)
                    pltpu.async_copy(col_hbm.at[wid, p], idx_c.at[p], isem)

            def zrow(r, carry):
                for c in range(d // 16):
                    buf[3, r, pl.ds(pl.multiple_of(c * 16, 16), 16)] = zero16
                return carry

            lax.fori_loop(0, CH, zrow, 0)
            pltpu.async_copy(table_hbm.at[pl.ds(base, RPT)],
                             tab_sh.at[pl.ds(base, RPT)], ssem)
            for k in range(RPT // CH):
                pltpu.async_copy(buf.at[3],
                                 acc_sh.at[pl.ds(base + k * CH, CH)], ssem)
            pltpu.make_async_copy(table_hbm.at[pl.ds(base, RPT)],
                                  tab_sh.at[pl.ds(base, RPT)], ssem).wait()
            for k in range(RPT // CH):
                pltpu.make_async_copy(buf.at[3],
                                      acc_sh.at[pl.ds(base, CH)], ssem).wait()
            plsc.subcore_barrier()

            for p in range(3):
                @pl.when(p < nch)
                def _():
                    pltpu.make_async_copy(row_hbm.at[wid, p], idx_r.at[p],
                                          isem).wait()
                    pltpu.make_async_copy(col_hbm.at[wid, p], idx_c.at[p],
                                          isem).wait()
                    pltpu.async_copy(tab_sh.at[idx_r.at[p]], buf.at[p], gsem)

            def body(ci, carry):
                @pl.when(ci >= 1)
                def _():
                    pltpu.make_async_copy(
                        buf.at[lax.rem(ci, 4)], acc_sh.at[idx_c.at[0]],
                        ssem).wait()

                @pl.when(ci + 7 < nch)
                def _():
                    s = lax.rem(ci + 7, 8)
                    pltpu.async_copy(row_hbm.at[wid, ci + 7], idx_r.at[s],
                                     isem)
                    pltpu.async_copy(col_hbm.at[wid, ci + 7], idx_c.at[s],
                                     isem)

                @pl.when(ci + 3 < nch)
                def _():
                    s = lax.rem(ci + 3, 8)
                    pltpu.make_async_copy(row_hbm.at[wid, 0], idx_r.at[0],
                                          isem).wait()
                    pltpu.make_async_copy(col_hbm.at[wid, 0], idx_c.at[0],
                                          isem).wait()
                    pltpu.async_copy(tab_sh.at[idx_r.at[s]],
                                     buf.at[lax.rem(ci + 3, 4)], gsem)

                pltpu.make_async_copy(tab_sh.at[idx_r.at[0]],
                                      buf.at[lax.rem(ci, 4)], gsem).wait()
                pltpu.async_copy(buf.at[lax.rem(ci, 4)],
                                 acc_sh.at[idx_c.at[lax.rem(ci, 8)]],
                                 ssem, add=True)
                return carry

            lax.fori_loop(0, nch, body, 0)

            @pl.when(nch >= 1)
            def _():
                pltpu.make_async_copy(buf.at[0], acc_sh.at[idx_c.at[0]],
                                      ssem).wait()
        else:
            idx_r, idx_c, buf, acc_sh, gsem, ssem = refs
            pltpu.sync_copy(row_hbm.at[wid], idx_r)
            pltpu.sync_copy(col_hbm.at[wid], idx_c)

            def zrow(r, carry):
                for c in range(d // 16):
                    buf[5, r, pl.ds(pl.multiple_of(c * 16, 16), 16)] = zero16
                return carry

            lax.fori_loop(0, CH, zrow, 0)
            for k in range(RPT // CH):
                pltpu.sync_copy(buf.at[5], acc_sh.at[pl.ds(base + k * CH, CH)])
            plsc.subcore_barrier()

            for p in range(5):
                @pl.when(p < nch)
                def _():
                    pltpu.async_copy(table_hbm.at[idx_r.at[p]], buf.at[p],
                                     gsem)

            def body(ci, carry):
                nxt = ci + 5

                @pl.when(ci >= 1)
                def _():
                    pltpu.make_async_copy(
                        buf.at[lax.rem(ci, 6)], acc_sh.at[idx_c.at[ci]],
                        ssem).wait()

                @pl.when(nxt < nch)
                def _():
                    pltpu.async_copy(table_hbm.at[idx_r.at[nxt]],
                                     buf.at[lax.rem(nxt, 6)], gsem)

                pltpu.make_async_copy(table_hbm.at[idx_r.at[ci]],
                                      buf.at[lax.rem(ci, 6)], gsem).wait()
                pltpu.async_copy(buf.at[lax.rem(ci, 6)],
                                 acc_sh.at[idx_c.at[ci]], ssem, add=True)
                return carry

            lax.fori_loop(0, nch, body, 0)

            @pl.when(nch >= 1)
            def _():
                pltpu.make_async_copy(buf.at[0], acc_sh.at[idx_c.at[0]],
                                      ssem).wait()

        plsc.subcore_barrier()
        pltpu.sync_copy(acc_sh.at[pl.ds(base, RPT)],
                        out_hbm.at[cid, pl.ds(base, RPT)])

    return edge_kernel(row_t, col_t, table)


def _tc_matmul(a, b):
    def mm_kernel(a_ref, b_ref, o_ref):
        o_ref[...] = jnp.dot(a_ref[...], b_ref[...],
                             preferred_element_type=jnp.float32)

    return pl.pallas_call(
        mm_kernel,
        out_shape=jax.ShapeDtypeStruct((a.shape[0], b.shape[1]), jnp.float32),
    )(a, b)


def _tc_norm_scale(deg_parts, xw):
    def k(p_ref, xw_ref, hp_ref, dinv_ref):
        deg = p_ref[0, :N] + p_ref[1, :N] + 1.0
        dinv = lax.rsqrt(deg)
        dinv_ref[...] = dinv
        hp_ref[...] = xw_ref[...] * dinv[:, None]

    return pl.pallas_call(
        k,
        out_shape=(
            jax.ShapeDtypeStruct((N, HID_DIM), jnp.float32),
            jax.ShapeDtypeStruct((N,), jnp.float32),
        ),
    )(deg_parts, xw)


def _tc_layer2_in(p1, hp1, dinv, W2, b1):
    def k(p_ref, hp_ref, dinv_ref, w_ref, b_ref, o_ref):
        s = p_ref[0, :N, :] + p_ref[1, :N, :] + hp_ref[...]
        dinv = dinv_ref[...]
        h1 = jnp.maximum(s * dinv[:, None] + b_ref[...], 0.0)
        o_ref[...] = jnp.dot(h1, w_ref[...],
                             preferred_element_type=jnp.float32) * dinv[:, None]

    return pl.pallas_call(
        k,
        out_shape=jax.ShapeDtypeStruct((N, EMB_DIM), jnp.float32),
    )(p1, hp1, dinv, W2, b1.reshape(1, HID_DIM))


def _tc_embed(p2, hp2, dinv, b2):
    def k(p_ref, hp_ref, dinv_ref, b_ref, o_ref):
        s = p_ref[0, :N, :] + p_ref[1, :N, :] + hp_ref[...]
        o_ref[...] = s * dinv_ref[...][:, None] + b_ref[...]

    return pl.pallas_call(
        k,
        out_shape=jax.ShapeDtypeStruct((N, EMB_DIM), jnp.float32),
    )(p2, hp2, dinv, b2.reshape(1, EMB_DIM))


def _tc_decode(z):
    BI, BJ = 512, 10240
    gi = pl.cdiv(N, BI)
    gj = pl.cdiv(N, BJ)

    def k(zi_ref, zj_ref, o_ref):
        g = lax.dot_general(zi_ref[...], zj_ref[...],
                            (((1,), (1,)), ((), ())),
                            preferred_element_type=jnp.float32)
        o_ref[...] = jax.nn.sigmoid(g)

    return pl.pallas_call(
        k,
        grid=(gi, gj),
        in_specs=[
            pl.BlockSpec((BI, EMB_DIM), lambda i, j: (i, 0)),
            pl.BlockSpec((BJ, EMB_DIM), lambda i, j: (j, 0)),
        ],
        out_specs=pl.BlockSpec((BI, BJ), lambda i, j: (i, j)),
        out_shape=jax.ShapeDtypeStruct((N, N), jnp.float32),
    )(z, z)


def _ragged(vals, pad_val, nchs):
    nch_max = max(nchs)
    cap0 = NUM_SUBCORES * nchs[0] * CH
    e0 = min(cap0, E)
    parts = []
    for lo, hi, nch in ((0, e0, nchs[0]), (e0, E, nchs[1])):
        cap = NUM_SUBCORES * nch * CH
        a = jnp.concatenate([
            vals[lo:hi],
            jnp.full((cap - (hi - lo),), pad_val, jnp.int32),
        ]) if cap else jnp.zeros((0,), jnp.int32)
        a = a.reshape(NUM_SUBCORES, nch, CH)
        a = jnp.concatenate([
            a, jnp.full((NUM_SUBCORES, nch_max - nch, CH), pad_val, jnp.int32)
        ], axis=1)
        parts.append(a)
    return jnp.stack(parts, axis=1).reshape(NW, nch_max, CH)


def kernel(x, edge_index, W1, b1, W2, b2):
    ei = edge_index.astype(jnp.int32)
    row_r = _ragged(ei[0], 0, NCH_EP)
    col_r = _ragged(ei[1], PAD_COL, NCH_EP)

    ones_vec = jnp.ones((CH,), jnp.float32)

    deg_parts = _sc_degree(col_r, ones_vec).reshape(NUM_CORES, NACC)
    xw = _tc_matmul(x, W1)

    hp1, dinv = _tc_norm_scale(deg_parts, xw)
    hp1_pad = jnp.concatenate(
        [hp1, jnp.zeros((NACC - N, HID_DIM), jnp.float32)])
    p1 = _sc_edge_pass(row_r, col_r, hp1_pad, HID_DIM, NCH_EP, stage=True)
    hp2 = _tc_layer2_in(p1, hp1, dinv, W2, b1)
    p2 = _sc_edge_pass(row_r, col_r, hp2, EMB_DIM, NCH_EP, stage=False)
    z = _tc_embed(p2, hp2, dinv, b2)
    return _tc_decode(z)


# speedup vs baseline: 1.0124x; 1.0124x over previous
import functools

import jax
import jax.numpy as jnp
from jax import lax
from jax.experimental import pallas as pl
from jax.experimental.pallas import tpu as pltpu
import jax.experimental.pallas.tpu_sc as plsc

N = 10000
E = 320000
IN_DIM = 128
HID_DIM = 64
EMB_DIM = 16

NUM_CORES = 2
NUM_SUBCORES = 16
NW = NUM_CORES * NUM_SUBCORES
CH = 128
NCH = 79
EPW = CH * NCH
E_PAD = NW * EPW
NCH_EP = (79, 79)
NACC = 10240
RPT = NACC // NUM_SUBCORES
PAD_COL = NACC - 1


def _sc_mesh():
    return plsc.VectorSubcoreMesh(
        core_axis_name="c", subcore_axis_name="s",
        num_cores=NUM_CORES, num_subcores=NUM_SUBCORES)


def _sc_degree(col_t, ones_vec):
    @functools.partial(
        pl.kernel,
        out_type=jax.ShapeDtypeStruct((NUM_CORES * NACC,), jnp.float32),
        mesh=_sc_mesh(),
        scratch_types=[
            pltpu.VMEM((NCH, CH), jnp.int32),
            pltpu.VMEM((CH,), jnp.float32),
            pltpu.VMEM((RPT,), jnp.float32),
            pltpu.VMEM_SHARED((NACC,), jnp.float32),
        ],
    )
    def deg_kernel(col_hbm, ones_hbm, out_hbm, idx_v, ones_v, zer_v, hist_sh):
        cid = lax.axis_index("c")
        sid = lax.axis_index("s")
        wid = sid * NUM_CORES + cid
        pltpu.sync_copy(col_hbm.at[wid], idx_v)
        pltpu.sync_copy(ones_hbm, ones_v)

        zero16 = jnp.zeros((16,), jnp.float32)

        def zbody(i, carry):
            zer_v[pl.ds(pl.multiple_of(i * 16, 16), 16)] = zero16
            return carry

        lax.fori_loop(0, RPT // 16, zbody, 0)
        base = pl.multiple_of(sid * RPT, 128)
        pltpu.sync_copy(zer_v, hist_sh.at[pl.ds(base, RPT)])
        plsc.subcore_barrier()

        def body(ci, carry):
            pltpu.sync_copy(ones_v, hist_sh.at[idx_v.at[ci]], add=True)
            return carry

        lax.fori_loop(0, NCH, body, 0)
        plsc.subcore_barrier()
        obase = pl.multiple_of(cid * NACC + sid * RPT, 128)
        pltpu.sync_copy(hist_sh.at[pl.ds(base, RPT)],
                        out_hbm.at[pl.ds(obase, RPT)])

    return deg_kernel(col_t, ones_vec)


def _sc_edge_pass(row_t, col_t, table, d, nchs, stage):
    nch_max = max(nchs)
    if stage:
        scratch = [
            pltpu.VMEM((8, CH), jnp.int32),
            pltpu.VMEM((8, CH), jnp.int32),
            pltpu.VMEM((4, CH, d), jnp.float32),
            pltpu.VMEM_SHARED((NACC, d), jnp.float32),
            pltpu.VMEM_SHARED((NACC, d), jnp.float32),
            pltpu.SemaphoreType.DMA,
            pltpu.SemaphoreType.DMA,
            pltpu.SemaphoreType.DMA,
        ]
    else:
        scratch = [
            pltpu.VMEM((nch_max, CH), jnp.int32),
            pltpu.VMEM((nch_max, CH), jnp.int32),
            pltpu.VMEM((6, CH, d), jnp.float32),
            pltpu.VMEM_SHARED((NACC, d), jnp.float32),
            pltpu.SemaphoreType.DMA,
            pltpu.SemaphoreType.DMA,
        ]

    @functools.partial(
        pl.kernel,
        out_type=jax.ShapeDtypeStruct((NUM_CORES, NACC, d), jnp.float32),
        mesh=_sc_mesh(),
        scratch_types=scratch,
        compiler_params=pltpu.CompilerParams(use_tc_tiling_on_sc=False),
    )
    def edge_kernel(row_hbm, col_hbm, table_hbm, out_hbm, *refs):
        cid = lax.axis_index("c")
        sid = lax.axis_index("s")
        wid = sid * NUM_CORES + cid
        nch = jnp.where(cid == 0, nchs[0], nchs[1])
        base = pl.multiple_of(sid * RPT, 8)
        zero16 = jnp.zeros((16,), jnp.float32)

        if stage:
            idx_r, idx_c, buf, acc_sh, tab_sh, gsem, ssem, isem = refs
            for p in range(7):
                @pl.when(p < nch)
                def _():
                    pltpu.async_copy(row_hbm.at[wid, p], idx_r.at[p], isem)
                    pltpu.async_copy(col_hbm.at[wid, p], idx_c.at[p], isem)

            def zrow(r, carry):
                for c in range(d // 16):
                    buf[3, r, pl.ds(pl.multiple_of(c * 16, 16), 16)] = zero16
                return carry

            lax.fori_loop(0, CH, zrow, 0)
            pltpu.async_copy(table_hbm.at[pl.ds(base, RPT)],
                             tab_sh.at[pl.ds(base, RPT)], ssem)
            for k in range(RPT // CH):
                pltpu.async_copy(buf.at[3],
                                 acc_sh.at[pl.ds(base + k * CH, CH)], ssem)
            pltpu.make_async_copy(table_hbm.at[pl.ds(base, RPT)],
                                  tab_sh.at[pl.ds(base, RPT)], ssem).wait()
            for k in range(RPT // CH):
                pltpu.make_async_copy(buf.at[3],
                                      acc_sh.at[pl.ds(base, CH)], ssem).wait()
            plsc.subcore_barrier()

            for p in range(3):
                @pl.when(p < nch)
                def _():
                    pltpu.make_async_copy(row_hbm.at[wid, p], idx_r.at[p],
                                          isem).wait()
                    pltpu.make_async_copy(col_hbm.at[wid, p], idx_c.at[p],
                                          isem).wait()
                    pltpu.async_copy(tab_sh.at[idx_r.at[p]], buf.at[p], gsem)

            def body(ci, carry):
                @pl.when(ci >= 1)
                def _():
                    pltpu.make_async_copy(
                        buf.at[lax.rem(ci, 4)], acc_sh.at[idx_c.at[0]],
                        ssem).wait()

                @pl.when(ci + 7 < nch)
                def _():
                    s = lax.rem(ci + 7, 8)
                    pltpu.async_copy(row_hbm.at[wid, ci + 7], idx_r.at[s],
                                     isem)
                    pltpu.async_copy(col_hbm.at[wid, ci + 7], idx_c.at[s],
                                     isem)

                @pl.when(ci + 3 < nch)
                def _():
                    s = lax.rem(ci + 3, 8)
                    pltpu.make_async_copy(row_hbm.at[wid, 0], idx_r.at[0],
                                          isem).wait()
                    pltpu.make_async_copy(col_hbm.at[wid, 0], idx_c.at[0],
                                          isem).wait()
                    pltpu.async_copy(tab_sh.at[idx_r.at[s]],
                                     buf.at[lax.rem(ci + 3, 4)], gsem)

                pltpu.make_async_copy(tab_sh.at[idx_r.at[0]],
                                      buf.at[lax.rem(ci, 4)], gsem).wait()
                pltpu.async_copy(buf.at[lax.rem(ci, 4)],
                                 acc_sh.at[idx_c.at[lax.rem(ci, 8)]],
                                 ssem, add=True)
                return carry

            lax.fori_loop(0, nch, body, 0)

            @pl.when(nch >= 1)
            def _():
                pltpu.make_async_copy(buf.at[0], acc_sh.at[idx_c.at[0]],
                                      ssem).wait()
        else:
            idx_r, idx_c, buf, acc_sh, gsem, ssem = refs
            pltpu.sync_copy(row_hbm.at[wid], idx_r)
            pltpu.sync_copy(col_hbm.at[wid], idx_c)

            def zrow(r, carry):
                for c in range(d // 16):
                    buf[5, r, pl.ds(pl.multiple_of(c * 16, 16), 16)] = zero16
                return carry

            lax.fori_loop(0, CH, zrow, 0)
            for k in range(RPT // CH):
                pltpu.sync_copy(buf.at[5], acc_sh.at[pl.ds(base + k * CH, CH)])
            plsc.subcore_barrier()

            for p in range(5):
                @pl.when(p < nch)
                def _():
                    pltpu.async_copy(table_hbm.at[idx_r.at[p]], buf.at[p],
                                     gsem)

            def body(ci, carry):
                nxt = ci + 5

                @pl.when(ci >= 1)
                def _():
                    pltpu.make_async_copy(
                        buf.at[lax.rem(ci, 6)], acc_sh.at[idx_c.at[ci]],
                        ssem).wait()

                @pl.when(nxt < nch)
                def _():
                    pltpu.async_copy(table_hbm.at[idx_r.at[nxt]],
                                     buf.at[lax.rem(nxt, 6)], gsem)

                pltpu.make_async_copy(table_hbm.at[idx_r.at[ci]],
                                      buf.at[lax.rem(ci, 6)], gsem).wait()
                pltpu.async_copy(buf.at[lax.rem(ci, 6)],
                                 acc_sh.at[idx_c.at[ci]], ssem, add=True)
                return carry

            lax.fori_loop(0, nch, body, 0)

            @pl.when(nch >= 1)
            def _():
                pltpu.make_async_copy(buf.at[0], acc_sh.at[idx_c.at[0]],
                                      ssem).wait()

        plsc.subcore_barrier()
        pltpu.sync_copy(acc_sh.at[pl.ds(base, RPT)],
                        out_hbm.at[cid, pl.ds(base, RPT)])

    return edge_kernel(row_t, col_t, table)


def _tc_matmul(a, b):
    def mm_kernel(a_ref, b_ref, o_ref):
        o_ref[...] = jnp.dot(a_ref[...], b_ref[...],
                             preferred_element_type=jnp.float32)

    return pl.pallas_call(
        mm_kernel,
        out_shape=jax.ShapeDtypeStruct((a.shape[0], b.shape[1]), jnp.float32),
    )(a, b)


def _tc_norm_scale(deg_parts, xw):
    def k(p_ref, xw_ref, hp_ref, dinv_ref):
        deg = p_ref[0, :N] + p_ref[1, :N] + 1.0
        dinv = lax.rsqrt(deg)
        dinv_ref[...] = dinv
        hp_ref[...] = xw_ref[...] * dinv[:, None]

    return pl.pallas_call(
        k,
        out_shape=(
            jax.ShapeDtypeStruct((N, HID_DIM), jnp.float32),
            jax.ShapeDtypeStruct((N,), jnp.float32),
        ),
    )(deg_parts, xw)


def _tc_layer2_in(p1, hp1, dinv, W2, b1):
    def k(p_ref, hp_ref, dinv_ref, w_ref, b_ref, o_ref):
        s = p_ref[0, :N, :] + p_ref[1, :N, :] + hp_ref[...]
        dinv = dinv_ref[...]
        h1 = jnp.maximum(s * dinv[:, None] + b_ref[...], 0.0)
        o_ref[...] = jnp.dot(h1, w_ref[...],
                             preferred_element_type=jnp.float32) * dinv[:, None]

    return pl.pallas_call(
        k,
        out_shape=jax.ShapeDtypeStruct((N, EMB_DIM), jnp.float32),
    )(p1, hp1, dinv, W2, b1.reshape(1, HID_DIM))


def _tc_embed(p2, hp2, dinv, b2):
    def k(p_ref, hp_ref, dinv_ref, b_ref, o_ref):
        s = p_ref[0, :N, :] + p_ref[1, :N, :] + hp_ref[...]
        o_ref[...] = s * dinv_ref[...][:, None] + b_ref[...]

    return pl.pallas_call(
        k,
        out_shape=jax.ShapeDtypeStruct((N, EMB_DIM), jnp.float32),
    )(p2, hp2, dinv, b2.reshape(1, EMB_DIM))


def _tc_decode(z):
    BI, BJ = 512, 10240
    gi = pl.cdiv(N, BI)
    gj = pl.cdiv(N, BJ)

    def k(zi_ref, zj_ref, o_ref):
        g = lax.dot_general(zi_ref[...], zj_ref[...],
                            (((1,), (1,)), ((), ())),
                            preferred_element_type=jnp.float32)
        o_ref[...] = jax.nn.sigmoid(g)

    return pl.pallas_call(
        k,
        grid=(gi, gj),
        in_specs=[
            pl.BlockSpec((BI, EMB_DIM), lambda i, j: (i, 0)),
            pl.BlockSpec((BJ, EMB_DIM), lambda i, j: (j, 0)),
        ],
        out_specs=pl.BlockSpec((BI, BJ), lambda i, j: (i, j)),
        out_shape=jax.ShapeDtypeStruct((N, N), jnp.float32),
    )(z, z)


def _ragged(vals, pad_val, nchs):
    nch_max = max(nchs)
    cap0 = NUM_SUBCORES * nchs[0] * CH
    e0 = min(cap0, E)
    parts = []
    for lo, hi, nch in ((0, e0, nchs[0]), (e0, E, nchs[1])):
        cap = NUM_SUBCORES * nch * CH
        a = jnp.concatenate([
            vals[lo:hi],
            jnp.full((cap - (hi - lo),), pad_val, jnp.int32),
        ]) if cap else jnp.zeros((0,), jnp.int32)
        a = a.reshape(NUM_SUBCORES, nch, CH)
        a = jnp.concatenate([
            a, jnp.full((NUM_SUBCORES, nch_max - nch, CH), pad_val, jnp.int32)
        ], axis=1)
        parts.append(a)
    return jnp.stack(parts, axis=1).reshape(NW, nch_max, CH)


def kernel(x, edge_index, W1, b1, W2, b2):
    ei = edge_index.astype(jnp.int32)
    pad = E_PAD - E
    col_t = jnp.concatenate(
        [ei[1], jnp.full((pad,), PAD_COL, jnp.int32)]).reshape(NW, NCH, CH)
    row_r = _ragged(ei[0], 0, NCH_EP)
    col_r = _ragged(ei[1], PAD_COL, NCH_EP)

    ones_vec = jnp.ones((CH,), jnp.float32)

    deg_parts = _sc_degree(col_t, ones_vec).reshape(NUM_CORES, NACC)
    xw = _tc_matmul(x, W1)

    hp1, dinv = _tc_norm_scale(deg_parts, xw)
    hp1_pad = jnp.concatenate(
        [hp1, jnp.zeros((NACC - N, HID_DIM), jnp.float32)])
    p1 = _sc_edge_pass(row_r, col_r, hp1_pad, HID_DIM, NCH_EP, stage=True)
    hp2 = _tc_layer2_in(p1, hp1, dinv, W2, b1)
    p2 = _sc_edge_pass(row_r, col_r, hp2, EMB_DIM, NCH_EP, stage=False)
    z = _tc_embed(p2, hp2, dinv, b2)
    return _tc_decode(z)
